# Initial kernel scaffold; baseline (speedup 1.0000x reference)
import jax
import jax.numpy as jnp
from jax.experimental import pallas as pl


def kernel(embed, indices):
    raise NotImplementedError("write your pallas kernel here")


# speedup vs baseline: 3.2254x; 3.2254x over previous
import jax
import jax.numpy as jnp
from jax import lax
from jax.experimental import pallas as pl
from jax.experimental.pallas import tpu as pltpu, tpu_sc as plsc

NC, NS = 2, 16
NW = NC * NS
R = 128
K = 8


def _gather_body(table_hbm, idx_hbm, out_hbm, idx_v, rows_v, sem):
    wid = lax.axis_index("s") * NC + lax.axis_index("c")
    nrows = idx_hbm.shape[0]
    per_w = nrows // NW
    base = wid * per_w

    def chunk(g, carry):
        off = base + g * K
        pltpu.sync_copy(idx_hbm.at[pl.ds(off, K)], idx_v)
        copies = [
            pltpu.async_copy(table_hbm.at[idx_v.at[j]], rows_v.at[j], sem)
            for j in range(K)
        ]
        for c in copies:
            c.wait()
        pltpu.sync_copy(rows_v, out_hbm.at[pl.ds(off, K)])
        return carry

    lax.fori_loop(0, per_w // K, chunk, 0)


def _embedding_gather(embed, idx2):
    nrows = idx2.shape[0]
    d = embed.shape[1]
    mesh = plsc.VectorSubcoreMesh(
        core_axis_name="c", subcore_axis_name="s",
        num_cores=NC, num_subcores=NS)
    return pl.kernel(
        _gather_body,
        out_type=jax.ShapeDtypeStruct((nrows, R, d), jnp.float32),
        mesh=mesh,
        scratch_types=[
            pltpu.VMEM((K, R), jnp.int32),
            pltpu.VMEM((K, R, d), jnp.float32),
            pltpu.SemaphoreType.DMA,
        ],
        compiler_params=pltpu.CompilerParams(use_tc_tiling_on_sc=False),
    )(embed, idx2)


def kernel(embed, indices):
    b, s = indices.shape
    idx2 = indices.astype(jnp.int32).reshape(-1, R)
    out = _embedding_gather(embed, idx2)
    return out.reshape(b, s, embed.shape[1])


# speedup vs baseline: 3.3204x; 1.0295x over previous
import jax
import jax.numpy as jnp
from jax import lax
from jax.experimental import pallas as pl
from jax.experimental.pallas import tpu as pltpu, tpu_sc as plsc

NC, NS = 2, 16
NW = NC * NS
R = 128
K = 4
NB = 4


def _gather_body(table_hbm, idx_hbm, out_hbm, idx_full, rows_v, *sems):
    gsem = sems[:NB]
    wsem = sems[NB:]
    wid = lax.axis_index("s") * NC + lax.axis_index("c")
    nrows = idx_hbm.shape[0]
    pw = nrows // NW
    nch = pw // K
    base = wid * pw

    pltpu.sync_copy(idx_hbm.at[pl.ds(base, pw)], idx_full)

    def fire_gathers(g, b):
        for k in range(K):
            pltpu.async_copy(table_hbm.at[idx_full.at[g * K + k]],
                             rows_v.at[b, k], gsem[b])

    def wait_gathers(b):
        for k in range(K):
            pltpu.make_async_copy(table_hbm.at[idx_full.at[0]],
                                  rows_v.at[b, k], gsem[b]).wait()

    def fire_write(g, b):
        pltpu.async_copy(rows_v.at[b], out_hbm.at[pl.ds(base + g * K, K)],
                         wsem[b])

    def wait_write(b):
        pltpu.make_async_copy(rows_v.at[b], out_hbm.at[pl.ds(base, K)],
                              wsem[b]).wait()

    fire_gathers(0, 0)
    for r in range(NB - 1):
        fire_gathers(r + 1, (r + 1) % NB)
        wait_gathers(r % NB)
        fire_write(r, r % NB)

    def main(go, carry):
        for bb in range(NB):
            g = (NB - 1) + go * NB + bb
            b = (bb + NB) % NB
            wait_write(b)
            fire_gathers(g + 1, b)
            wait_gathers((bb + NB - 1) % NB)
            fire_write(g, (bb + NB - 1) % NB)
        return carry

    n_rounds = (nch - NB) // NB
    lax.fori_loop(0, n_rounds, main, 0, unroll=False)

    for g in range((NB - 1) + n_rounds * NB, nch - 1):
        b = (g + 1) % NB
        wait_write(b)
        fire_gathers(g + 1, b)
        wait_gathers(g % NB)
        fire_write(g, g % NB)

    wait_gathers((nch - 1) % NB)
    fire_write(nch - 1, (nch - 1) % NB)
    for b in range(NB):
        wait_write((nch - NB + b) % NB)


def _embedding_gather(embed, idx2):
    nrows = idx2.shape[0]
    d = embed.shape[1]
    pw = nrows // NW
    mesh = plsc.VectorSubcoreMesh(
        core_axis_name="c", subcore_axis_name="s",
        num_cores=NC, num_subcores=NS)
    return pl.kernel(
        _gather_body,
        out_type=jax.ShapeDtypeStruct((nrows, R, d), jnp.float32),
        mesh=mesh,
        scratch_types=(
            [pltpu.VMEM((pw, R), jnp.int32),
             pltpu.VMEM((NB, K, R, d), jnp.float32)]
            + [pltpu.SemaphoreType.DMA] * (2 * NB)
        ),
        compiler_params=pltpu.CompilerParams(use_tc_tiling_on_sc=False),
    )(embed, idx2)


def kernel(embed, indices):
    b, s = indices.shape
    idx2 = indices.astype(jnp.int32).reshape(-1, R)
    out = _embedding_gather(embed, idx2)
    return out.reshape(b, s, embed.shape[1])


# speedup vs baseline: 4.4365x; 1.3361x over previous
import jax
import jax.numpy as jnp
from jax import lax
from jax.experimental import pallas as pl
from jax.experimental.pallas import tpu as pltpu, tpu_sc as plsc

NC, NS = 2, 16
NW = NC * NS
R = 128
K = 8
NB = 2


def _gather_body(table_hbm, idx_hbm, out_hbm, idx_full, rows_v, *sems):
    gsem = sems[:NB]
    wsem = sems[NB:]
    wid = lax.axis_index("s") * NC + lax.axis_index("c")
    nrows = idx_hbm.shape[0]
    pw = nrows // NW
    nch = pw // K
    base = wid * pw

    pltpu.sync_copy(idx_hbm.at[pl.ds(base, pw)], idx_full)

    def fire_gathers(g, b):
        for k in range(K):
            pltpu.async_copy(table_hbm.at[idx_full.at[g * K + k]],
                             rows_v.at[b, k], gsem[b])

    def wait_gathers(b):
        for k in range(K):
            pltpu.make_async_copy(table_hbm.at[idx_full.at[0]],
                                  rows_v.at[b, k], gsem[b]).wait()

    def fire_write(g, b):
        pltpu.async_copy(rows_v.at[b], out_hbm.at[pl.ds(base + g * K, K)],
                         wsem[b])

    def wait_write(b):
        pltpu.make_async_copy(rows_v.at[b], out_hbm.at[pl.ds(base, K)],
                              wsem[b]).wait()

    fire_gathers(0, 0)
    for r in range(NB - 1):
        fire_gathers(r + 1, (r + 1) % NB)
        wait_gathers(r % NB)
        fire_write(r, r % NB)

    def main(go, carry):
        for bb in range(NB):
            g = (NB - 1) + go * NB + bb
            b = (bb + NB) % NB
            wait_write(b)
            fire_gathers(g + 1, b)
            wait_gathers((bb + NB - 1) % NB)
            fire_write(g, (bb + NB - 1) % NB)
        return carry

    n_rounds = (nch - NB) // NB
    lax.fori_loop(0, n_rounds, main, 0, unroll=False)

    for g in range((NB - 1) + n_rounds * NB, nch - 1):
        b = (g + 1) % NB
        wait_write(b)
        fire_gathers(g + 1, b)
        wait_gathers(g % NB)
        fire_write(g, g % NB)

    wait_gathers((nch - 1) % NB)
    fire_write(nch - 1, (nch - 1) % NB)
    for b in range(NB):
        wait_write((nch - NB + b) % NB)


def _embedding_gather(embed, idx2):
    nrows = idx2.shape[0]
    d = embed.shape[1]
    pw = nrows // NW
    mesh = plsc.VectorSubcoreMesh(
        core_axis_name="c", subcore_axis_name="s",
        num_cores=NC, num_subcores=NS)
    return pl.kernel(
        _gather_body,
        out_type=jax.ShapeDtypeStruct((nrows, R, d), jnp.float32),
        mesh=mesh,
        scratch_types=(
            [pltpu.VMEM((pw, R), jnp.int32),
             pltpu.VMEM((NB, K, R, d), jnp.float32)]
            + [pltpu.SemaphoreType.DMA] * (2 * NB)
        ),
        compiler_params=pltpu.CompilerParams(use_tc_tiling_on_sc=False),
    )(embed, idx2)


def kernel(embed, indices):
    b, s = indices.shape
    idx2 = indices.astype(jnp.int32).reshape(-1, R)
    out = _embedding_gather(embed, idx2)
    return out.reshape(b, s, embed.shape[1])
